# baseline (speedup 1.0000x reference)
import jax
import jax.numpy as jnp
from jax.experimental import pallas as pl
from jax.experimental.pallas import tpu as pltpu


def _transpose_kernel(f_ref, o_ref, s_ref, sem):
    b = pl.program_id(0)
    c, n = f_ref.shape[1], f_ref.shape[2]
    s_ref.reshape(n, c)[...] = f_ref[0].T
    copy = pltpu.make_async_copy(
        s_ref, o_ref.at[b].reshape(n // 2, 2, c), sem
    )
    copy.start()
    copy.wait()


def kernel(xyz, xyz_fp, features, features_fp, W, b):
    B, C, N = features.shape
    out = pl.pallas_call(
        _transpose_kernel,
        grid=(B,),
        in_specs=[pl.BlockSpec((1, C, N), lambda i: (i, 0, 0))],
        out_specs=pl.BlockSpec(memory_space=pltpu.MemorySpace.HBM),
        out_shape=jax.ShapeDtypeStruct((B, N, C), features.dtype),
        scratch_shapes=[
            pltpu.VMEM((N // 2, 2, C), jnp.float32),
            pltpu.SemaphoreType.DMA,
        ],
    )(features)
    return out
